# Initial kernel scaffold; baseline (speedup 1.0000x reference)
import jax
import jax.numpy as jnp
from jax.experimental import pallas as pl


def kernel(x, pos, edge_index_local, edge_index_global, batch, W_atom, b_atom, W1, b1, W2, b2, W_upd, b_upd, ln_g, ln_b, W_lat, b_lat, Wn1, bn1, Wn2, bn2, Wg1, bg1, Wg2, bg2):
    raise NotImplementedError("write your pallas kernel here")


# speedup vs baseline: 1.5725x; 1.5725x over previous
import functools

import jax
import jax.numpy as jnp
from jax import lax
from jax.experimental import pallas as pl
from jax.experimental.pallas import tpu as pltpu
from jax.experimental.pallas import tpu_sc as plsc

F32 = jnp.float32
N = 10000
NG = 256
SDIM = 128
RBF = 32
LATENT = 128
CUTOFF = 7.5
LAYERS = 5
E = 320000

NCORE = 2
NSUB = 16
NW = NCORE * NSUB
CH = 128
EPW = 10240
NCHUNK = EPW // CH
E_PAD = NW * EPW
N_PAD = 10240
TR = N_PAD // NSUB

NB = 1000
NBLK = N // NB
BE = 2048
NEB = E_PAD // BE

@functools.lru_cache(maxsize=None)
def _mesh():
    return plsc.VectorSubcoreMesh(core_axis_name="c", subcore_axis_name="s",
                                  num_cores=NCORE, num_subcores=NSUB)


@functools.lru_cache(maxsize=None)
def _gather_pair_kernel(D):
    @functools.partial(
        pl.kernel,
        mesh=_mesh(),
        out_type=(jax.ShapeDtypeStruct((E_PAD, D), F32),
                  jax.ShapeDtypeStruct((E_PAD, D), F32)),
        scratch_types=[
            pltpu.VMEM((CH,), jnp.int32),
            pltpu.VMEM((CH,), jnp.int32),
            pltpu.VMEM((CH, D), F32),
            pltpu.VMEM((CH, D), F32),
            pltpu.SemaphoreType.DMA,
            pltpu.SemaphoreType.DMA,
        ],
        name=f"sc_gather_pair_{D}",
    )
    def gk(t1, t2, idx1, idx2, o1, o2, i1v, i2v, b1v, b2v, s1, s2):
        wid = lax.axis_index("s") * NCORE + lax.axis_index("c")

        def body(j, carry):
            base = wid * EPW + j * CH
            pltpu.sync_copy(idx1.at[wid, j], i1v)
            pltpu.sync_copy(idx2.at[wid, j], i2v)
            c1 = pltpu.async_copy(t1.at[i1v], b1v, s1)
            c2 = pltpu.async_copy(t2.at[i2v], b2v, s2)
            c1.wait()
            c2.wait()
            pltpu.sync_copy(b1v, o1.at[pl.ds(base, CH)])
            pltpu.sync_copy(b2v, o2.at[pl.ds(base, CH)])
            return carry

        lax.fori_loop(0, NCHUNK, body, 0)

    return gk


@functools.lru_cache(maxsize=None)
def _scatter_add_kernel():
    @functools.partial(
        pl.kernel,
        mesh=_mesh(),
        out_type=jax.ShapeDtypeStruct((NCORE, N_PAD, SDIM), F32),
        scratch_types=[
            pltpu.VMEM((CH,), jnp.int32),
            pltpu.VMEM((CH, SDIM), F32),
            pltpu.VMEM_SHARED((N_PAD, SDIM), F32),
        ],
        name="sc_scatter_add",
    )
    def sk(ms, didx, zb, out, iv, bv, acc):
        cid = lax.axis_index("c")
        sid = lax.axis_index("s")
        wid = sid * NCORE + cid
        pltpu.sync_copy(zb, acc.at[pl.ds(sid * TR, TR)])
        plsc.subcore_barrier()

        def body(j, carry):
            base = wid * EPW + j * CH
            pltpu.sync_copy(didx.at[wid, j], iv)
            pltpu.sync_copy(ms.at[pl.ds(base, CH)], bv)
            pltpu.sync_copy(bv, acc.at[iv], add=True)
            return carry

        lax.fori_loop(0, NCHUNK, body, 0)
        plsc.subcore_barrier()
        pltpu.sync_copy(acc.at[pl.ds(sid * TR, TR)],
                        out.at[cid, pl.ds(sid * TR, TR)])

    return sk


def _sc_gather_pair(t1, t2, idx1, idx2):
    return _gather_pair_kernel(t1.shape[1])(t1, t2, idx1, idx2)


def _sc_scatter_add(ms, didx, zb):
    return _scatter_add_kernel()(ms, didx, zb)


def _sigmoid(x):
    return 1.0 / (1.0 + jnp.exp(-x))


def _silu(x):
    return x * _sigmoid(x)


def _pos_stats(pos, batch_rows):
    def body(pos_ref, bat_ref, ps_ref, cnt_ref):
        @pl.when(pl.program_id(0) == 0)
        def _():
            ps_ref[...] = jnp.zeros_like(ps_ref)
            cnt_ref[...] = jnp.zeros_like(cnt_ref)

        bat = bat_ref[...].reshape(1, NB)
        oh = (lax.broadcasted_iota(jnp.int32, (NG, NB), 0) == bat).astype(F32)
        ps_ref[...] += oh @ pos_ref[...]
        cnt_ref[...] += jnp.sum(oh, axis=1, keepdims=True)

    return pl.pallas_call(
        body,
        grid=(NBLK,),
        in_specs=[pl.BlockSpec((NB, 3), lambda i: (i, 0)),
                  pl.BlockSpec((1, 1, NB), lambda i: (i, 0, 0))],
        out_specs=(pl.BlockSpec((NG, 3), lambda i: (0, 0)),
                   pl.BlockSpec((NG, 1), lambda i: (0, 0))),
        out_shape=(jax.ShapeDtypeStruct((NG, 3), F32),
                   jax.ShapeDtypeStruct((NG, 1), F32)),
    )(pos, batch_rows)


def _init_nodes(x, pos, batch_col, ps, cnt, W_atom, b_atom, W1a, W1b):
    def body(x_ref, pos_ref, bat_ref, ps_ref, cnt_ref, wa_ref, ba_ref,
             wp_ref, wq_ref, s_ref, p_ref, q_ref, pp_ref):
        mean = ps_ref[...] / jnp.maximum(cnt_ref[...], 1.0)
        oh = (lax.broadcasted_iota(jnp.int32, (NB, NG), 1)
              == bat_ref[...]).astype(F32)
        cpos = pos_ref[...] - oh @ mean
        pp_ref[...] = jnp.concatenate(
            [cpos, jnp.zeros((NB, SDIM - 3), F32)], axis=1)
        s0 = x_ref[...] @ wa_ref[...] + ba_ref[...]
        s_ref[...] = s0
        p_ref[...] = s0 @ wp_ref[...]
        q_ref[...] = s0 @ wq_ref[...]

    return pl.pallas_call(
        body,
        grid=(NBLK,),
        in_specs=[
            pl.BlockSpec((NB, 16), lambda i: (i, 0)),
            pl.BlockSpec((NB, 3), lambda i: (i, 0)),
            pl.BlockSpec((NB, 1), lambda i: (i, 0)),
            pl.BlockSpec((NG, 3), lambda i: (0, 0)),
            pl.BlockSpec((NG, 1), lambda i: (0, 0)),
            pl.BlockSpec((16, SDIM), lambda i: (0, 0)),
            pl.BlockSpec((1, SDIM), lambda i: (0, 0)),
            pl.BlockSpec((SDIM, SDIM), lambda i: (0, 0)),
            pl.BlockSpec((SDIM, SDIM), lambda i: (0, 0)),
        ],
        out_specs=(pl.BlockSpec((NB, SDIM), lambda i: (i, 0)),
                   pl.BlockSpec((NB, SDIM), lambda i: (i, 0)),
                   pl.BlockSpec((NB, SDIM), lambda i: (i, 0)),
                   pl.BlockSpec((NB, SDIM), lambda i: (i, 0))),
        out_shape=(jax.ShapeDtypeStruct((N, SDIM), F32),
                   jax.ShapeDtypeStruct((N, SDIM), F32),
                   jax.ShapeDtypeStruct((N, SDIM), F32),
                   jax.ShapeDtypeStruct((N, SDIM), F32)),
    )(x, pos, batch_col, ps, cnt, W_atom, b_atom, W1a, W1b)


def _edge_geometry(posS, posD):
    def body(ps_ref, pd_ref, out_ref):
        s3 = ps_ref[...][:, 0:3]
        d3 = pd_ref[...][:, 0:3]
        r = d3 - s3
        d2 = jnp.sum(r * r, axis=1, keepdims=True)
        aa = jnp.sum(d3 * s3, axis=1, keepdims=True)
        dd = jnp.sqrt(jnp.maximum(d2, 1e-6))
        step = CUTOFF / (RBF - 1)
        centers = lax.broadcasted_iota(jnp.int32, (1, RBF), 1).astype(F32) * step
        gamma = (RBF / CUTOFF) ** 2
        rbf = jnp.exp(-gamma * (dd - centers) ** 2)
        out_ref[...] = jnp.concatenate(
            [rbf, aa, jnp.zeros((BE, 7), F32)], axis=1)

    return pl.pallas_call(
        body,
        grid=(NEB,),
        in_specs=[pl.BlockSpec((BE, SDIM), lambda i: (i, 0)),
                  pl.BlockSpec((BE, SDIM), lambda i: (i, 0))],
        out_specs=pl.BlockSpec((BE, 40), lambda i: (i, 0)),
        out_shape=jax.ShapeDtypeStruct((E_PAD, 40), F32),
    )(posS, posD)


def _edge_mlp(gs, gd, rbfa, W1cd, b1, W2s, b2s):
    def body(gs_ref, gd_ref, rb_ref, w1_ref, b1_ref, w2_ref, b2_ref, out_ref):
        pre = gs_ref[...] + gd_ref[...] + rb_ref[...] @ w1_ref[...] + b1_ref[...]
        h = _silu(pre)
        m = h @ w2_ref[...] + b2_ref[...]
        row = (pl.program_id(0) * BE
               + lax.broadcasted_iota(jnp.int32, (BE, 1), 0))
        out_ref[...] = jnp.where(row < E, m, 0.0)

    return pl.pallas_call(
        body,
        grid=(NEB,),
        in_specs=[
            pl.BlockSpec((BE, SDIM), lambda i: (i, 0)),
            pl.BlockSpec((BE, SDIM), lambda i: (i, 0)),
            pl.BlockSpec((BE, 40), lambda i: (i, 0)),
            pl.BlockSpec((40, SDIM), lambda i: (0, 0)),
            pl.BlockSpec((1, SDIM), lambda i: (0, 0)),
            pl.BlockSpec((SDIM, SDIM), lambda i: (0, 0)),
            pl.BlockSpec((1, SDIM), lambda i: (0, 0)),
        ],
        out_specs=pl.BlockSpec((BE, SDIM), lambda i: (i, 0)),
        out_shape=jax.ShapeDtypeStruct((E_PAD, SDIM), F32),
    )(gs, gd, rbfa, W1cd, b1, W2s, b2s)


def _node_update(s, agg0, agg1, Wu, bu, g, bb, Wp, Wq):
    def body(s_ref, a0_ref, a1_ref, wu_ref, bu_ref, g_ref, bb_ref,
             wp_ref, wq_ref, sn_ref, p_ref, q_ref):
        u = (s_ref[...] + (a0_ref[...] + a1_ref[...]) @ wu_ref[...]
             + bu_ref[...])
        mu = jnp.mean(u, axis=1, keepdims=True)
        var = jnp.mean((u - mu) ** 2, axis=1, keepdims=True)
        sn = (u - mu) / jnp.sqrt(var + 1e-5) * g_ref[...] + bb_ref[...]
        sn_ref[...] = sn
        p_ref[...] = sn @ wp_ref[...]
        q_ref[...] = sn @ wq_ref[...]

    return pl.pallas_call(
        body,
        grid=(NBLK,),
        in_specs=[
            pl.BlockSpec((NB, SDIM), lambda i: (i, 0)),
            pl.BlockSpec((NB, SDIM), lambda i: (i, 0)),
            pl.BlockSpec((NB, SDIM), lambda i: (i, 0)),
            pl.BlockSpec((SDIM, SDIM), lambda i: (0, 0)),
            pl.BlockSpec((1, SDIM), lambda i: (0, 0)),
            pl.BlockSpec((1, SDIM), lambda i: (0, 0)),
            pl.BlockSpec((1, SDIM), lambda i: (0, 0)),
            pl.BlockSpec((SDIM, SDIM), lambda i: (0, 0)),
            pl.BlockSpec((SDIM, SDIM), lambda i: (0, 0)),
        ],
        out_specs=(pl.BlockSpec((NB, SDIM), lambda i: (i, 0)),
                   pl.BlockSpec((NB, SDIM), lambda i: (i, 0)),
                   pl.BlockSpec((NB, SDIM), lambda i: (i, 0))),
        out_shape=(jax.ShapeDtypeStruct((N, SDIM), F32),
                   jax.ShapeDtypeStruct((N, SDIM), F32),
                   jax.ShapeDtypeStruct((N, SDIM), F32)),
    )(s, agg0, agg1, Wu, bu, g, bb, Wp, Wq)


def _head_mlps(s, W_lat, b_lat, Wg1, bg1, Wg2, bg2, Wn1, bn1, Wn2, bn2):
    def body(s_ref, wl, bl, wg1, bg1_, wg2, bg2_, wn1, bn1_, wn2, bn2_,
             gl_ref, node_ref):
        out = s_ref[...] @ wl[...] + bl[...]
        hg = _silu(out @ wg1[...] + bg1_[...])
        gl_ref[...] = hg @ wg2[...] + bg2_[...]
        hn = _silu(out @ wn1[...] + bn1_[...])
        node_ref[...] = hn @ wn2[...] + bn2_[...]

    return pl.pallas_call(
        body,
        grid=(NBLK,),
        in_specs=[
            pl.BlockSpec((NB, SDIM), lambda i: (i, 0)),
            pl.BlockSpec((SDIM, LATENT), lambda i: (0, 0)),
            pl.BlockSpec((1, LATENT), lambda i: (0, 0)),
            pl.BlockSpec((LATENT, LATENT), lambda i: (0, 0)),
            pl.BlockSpec((1, LATENT), lambda i: (0, 0)),
            pl.BlockSpec((LATENT, 1), lambda i: (0, 0)),
            pl.BlockSpec((1, 1), lambda i: (0, 0)),
            pl.BlockSpec((LATENT, LATENT), lambda i: (0, 0)),
            pl.BlockSpec((1, LATENT), lambda i: (0, 0)),
            pl.BlockSpec((LATENT, LATENT), lambda i: (0, 0)),
            pl.BlockSpec((1, LATENT), lambda i: (0, 0)),
        ],
        out_specs=(pl.BlockSpec((NB, 1), lambda i: (i, 0)),
                   pl.BlockSpec((NB, LATENT), lambda i: (i, 0))),
        out_shape=(jax.ShapeDtypeStruct((N, 1), F32),
                   jax.ShapeDtypeStruct((N, LATENT), F32)),
    )(s, W_lat, b_lat, Wg1, bg1, Wg2, bg2, Wn1, bn1, Wn2, bn2)


def _gate_max(gl, batch_col):
    def body(gl_ref, bat_ref, gm_ref):
        @pl.when(pl.program_id(0) == 0)
        def _():
            gm_ref[...] = jnp.full_like(gm_ref, -jnp.inf)

        oh = (lax.broadcasted_iota(jnp.int32, (NB, NG), 1) == bat_ref[...])
        masked = jnp.where(oh, gl_ref[...], -jnp.inf)
        gm_ref[...] = jnp.maximum(gm_ref[...],
                                  jnp.max(masked, axis=0, keepdims=True))

    return pl.pallas_call(
        body,
        grid=(NBLK,),
        in_specs=[pl.BlockSpec((NB, 1), lambda i: (i, 0)),
                  pl.BlockSpec((NB, 1), lambda i: (i, 0))],
        out_specs=pl.BlockSpec((1, NG), lambda i: (0, 0)),
        out_shape=jax.ShapeDtypeStruct((1, NG), F32),
    )(gl, batch_col)


def _pool(gl, node, batch_col, batch_rows, gmax):
    def body(gl_ref, node_ref, bat_ref, batr_ref, gm_ref, num_ref, gs_ref):
        @pl.when(pl.program_id(0) == 0)
        def _():
            num_ref[...] = jnp.zeros_like(num_ref)
            gs_ref[...] = jnp.zeros_like(gs_ref)

        oh = (lax.broadcasted_iota(jnp.int32, (NB, NG), 1) == bat_ref[...])
        gmax_g = jnp.sum(jnp.where(oh, gm_ref[...], 0.0), axis=1,
                         keepdims=True)
        eg = jnp.exp(gl_ref[...] - gmax_g)
        batr = batr_ref[...].reshape(1, NB)
        ohT = (lax.broadcasted_iota(jnp.int32, (NG, NB), 0)
               == batr).astype(F32)
        num_ref[...] += ohT @ (eg * node_ref[...])
        gs_ref[...] += ohT @ eg

    return pl.pallas_call(
        body,
        grid=(NBLK,),
        in_specs=[
            pl.BlockSpec((NB, 1), lambda i: (i, 0)),
            pl.BlockSpec((NB, LATENT), lambda i: (i, 0)),
            pl.BlockSpec((NB, 1), lambda i: (i, 0)),
            pl.BlockSpec((1, 1, NB), lambda i: (i, 0, 0)),
            pl.BlockSpec((1, NG), lambda i: (0, 0)),
        ],
        out_specs=(pl.BlockSpec((NG, LATENT), lambda i: (0, 0)),
                   pl.BlockSpec((NG, 1), lambda i: (0, 0))),
        out_shape=(jax.ShapeDtypeStruct((NG, LATENT), F32),
                   jax.ShapeDtypeStruct((NG, 1), F32)),
    )(gl, node, batch_col, batch_rows, gmax)


def _finalize(num, gs):
    def body(num_ref, gs_ref, out_ref):
        out_ref[...] = num_ref[...] / (gs_ref[...] + 1e-16)

    return pl.pallas_call(
        body,
        in_specs=[pl.BlockSpec((NG, LATENT), lambda: (0, 0)),
                  pl.BlockSpec((NG, 1), lambda: (0, 0))],
        out_specs=pl.BlockSpec((NG, LATENT), lambda: (0, 0)),
        out_shape=jax.ShapeDtypeStruct((NG, LATENT), F32),
    )(num, gs)


def _prep_idx(idx):
    p = jnp.zeros((E_PAD,), jnp.int32).at[:E].set(idx)
    return p.reshape(NW, NCHUNK, CH)


def kernel(x, pos, edge_index_local, edge_index_global, batch, W_atom,
           b_atom, W1, b1, W2, b2, W_upd, b_upd, ln_g, ln_b, W_lat, b_lat,
           Wn1, bn1, Wn2, bn2, Wg1, bg1, Wg2, bg2):
    batch_col = batch.reshape(N, 1)
    batch_rows = batch.reshape(NBLK, 1, NB)
    row = lambda v: v.reshape(1, -1)

    srcR = [_prep_idx(edge_index_local[0]), _prep_idx(edge_index_global[0])]
    dstR = [_prep_idx(edge_index_local[1]), _prep_idx(edge_index_global[1])]

    ps, cnt = _pos_stats(pos, batch_rows)
    s, P, Q, pos128 = _init_nodes(
        x, pos, batch_col, ps, cnt, W_atom, row(b_atom),
        W1[0, 0, :SDIM, :], W1[0, 0, SDIM:2 * SDIM, :])

    rbfa = []
    for j in range(2):
        posS, posD = _sc_gather_pair(pos128, pos128, srcR[j], dstR[j])
        rbfa.append(_edge_geometry(posS, posD))

    zb = jnp.zeros((TR, SDIM), F32)

    for p in range(2 * LAYERS):
        l, j = p // 2, p % 2
        W1cd = jnp.zeros((40, SDIM), F32).at[:33].set(W1[l, j, 2 * SDIM:, :])
        gs, gd = _sc_gather_pair(P, Q, srcR[j], dstR[j])
        ms = _edge_mlp(gs, gd, rbfa[j], W1cd, row(b1[l, j]),
                       W2[l, j][:, :SDIM], row(b2[l, j][:SDIM]))
        agg = _sc_scatter_add(ms, dstR[j], zb)
        a0, a1 = agg[0, :N], agg[1, :N]
        ln_, jn_ = (p + 1) // 2, (p + 1) % 2
        if p == 2 * LAYERS - 1:
            ln_, jn_ = 0, 0
        s, P, Q = _node_update(
            s, a0, a1, W_upd[l, j], row(b_upd[l, j]),
            row(ln_g[l, j]), row(ln_b[l, j]),
            W1[ln_, jn_, :SDIM, :], W1[ln_, jn_, SDIM:2 * SDIM, :])

    gl, node = _head_mlps(s, W_lat, row(b_lat), Wg1, row(bg1),
                          Wg2, bg2.reshape(1, 1), Wn1, row(bn1),
                          Wn2, row(bn2))
    gmax = _gate_max(gl, batch_col)
    num, gs_ = _pool(gl, node, batch_col, batch_rows, gmax)
    return _finalize(num, gs_)


# speedup vs baseline: 1.9598x; 1.2463x over previous
import functools

import jax
import jax.numpy as jnp
from jax import lax
from jax.experimental import pallas as pl
from jax.experimental.pallas import tpu as pltpu
from jax.experimental.pallas import tpu_sc as plsc

F32 = jnp.float32
N = 10000
NG = 256
SDIM = 128
RBF = 32
LATENT = 128
CUTOFF = 7.5
LAYERS = 5
E = 320000

NCORE = 2
NSUB = 16
NW = NCORE * NSUB
CH = 128
EPW = 10240
NCHUNK = EPW // CH
E_PAD = NW * EPW
N_PAD = 10240
TR = N_PAD // NSUB

NB = 1000
NBLK = N // NB
BE = 2048
NEB = E_PAD // BE

@functools.lru_cache(maxsize=None)
def _mesh():
    return plsc.VectorSubcoreMesh(core_axis_name="c", subcore_axis_name="s",
                                  num_cores=NCORE, num_subcores=NSUB)


@functools.lru_cache(maxsize=None)
def _gather_pair_kernel(D):
    @functools.partial(
        pl.kernel,
        mesh=_mesh(),
        out_type=(jax.ShapeDtypeStruct((E_PAD, D), F32),
                  jax.ShapeDtypeStruct((E_PAD, D), F32)),
        scratch_types=[
            pltpu.VMEM((NCHUNK, CH), jnp.int32),
            pltpu.VMEM((NCHUNK, CH), jnp.int32),
            pltpu.VMEM((2, CH, D), F32),
            pltpu.VMEM((2, CH, D), F32),
            pltpu.SemaphoreType.DMA((2,)),
            pltpu.SemaphoreType.DMA((2,)),
            pltpu.SemaphoreType.DMA((2,)),
            pltpu.SemaphoreType.DMA((2,)),
        ],
        name=f"sc_gather_pair_{D}",
    )
    def gk(t1, t2, idx1, idx2, o1, o2, i1a, i2a, b1, b2, sg1, sg2, sw1, sw2):
        wid = lax.axis_index("s") * NCORE + lax.axis_index("c")
        pltpu.sync_copy(idx1.at[wid], i1a)
        pltpu.sync_copy(idx2.at[wid], i2a)

        def start_gather(j, slot):
            pltpu.async_copy(t1.at[i1a.at[j]], b1.at[slot], sg1.at[slot])
            pltpu.async_copy(t2.at[i2a.at[j]], b2.at[slot], sg2.at[slot])

        def wait_gather(slot):
            pltpu.make_async_copy(t1.at[pl.ds(0, CH)], b1.at[slot],
                                  sg1.at[slot]).wait()
            pltpu.make_async_copy(t2.at[pl.ds(0, CH)], b2.at[slot],
                                  sg2.at[slot]).wait()

        def start_write(j, slot):
            base = wid * EPW + j * CH
            pltpu.async_copy(b1.at[slot], o1.at[pl.ds(base, CH)], sw1.at[slot])
            pltpu.async_copy(b2.at[slot], o2.at[pl.ds(base, CH)], sw2.at[slot])

        def wait_write(slot):
            pltpu.make_async_copy(b1.at[slot], o1.at[pl.ds(0, CH)],
                                  sw1.at[slot]).wait()
            pltpu.make_async_copy(b2.at[slot], o2.at[pl.ds(0, CH)],
                                  sw2.at[slot]).wait()

        start_gather(0, 0)

        def body(j, carry):
            slot = lax.rem(j, 2)
            prev = 1 - slot

            @pl.when(j >= 2)
            def _():
                wait_write(slot)

            start_gather(j, slot)
            wait_gather(prev)
            start_write(j - 1, prev)
            return carry

        lax.fori_loop(1, NCHUNK, body, 0)
        last = (NCHUNK - 1) % 2
        wait_gather(last)
        start_write(NCHUNK - 1, last)
        wait_write(last)
        wait_write(1 - last)

    return gk


@functools.lru_cache(maxsize=None)
def _scatter_add_kernel():
    @functools.partial(
        pl.kernel,
        mesh=_mesh(),
        out_type=jax.ShapeDtypeStruct((NCORE, N_PAD, SDIM), F32),
        scratch_types=[
            pltpu.VMEM((NCHUNK, CH), jnp.int32),
            pltpu.VMEM((2, CH, SDIM), F32),
            pltpu.VMEM_SHARED((N_PAD, SDIM), F32),
            pltpu.SemaphoreType.DMA((2,)),
            pltpu.SemaphoreType.DMA((2,)),
        ],
        name="sc_scatter_add",
    )
    def sk(ms, didx, zb, out, ia, bv, acc, sr, sa):
        cid = lax.axis_index("c")
        sid = lax.axis_index("s")
        wid = sid * NCORE + cid
        pltpu.sync_copy(zb, acc.at[pl.ds(sid * TR, TR)])
        pltpu.sync_copy(didx.at[wid], ia)
        plsc.subcore_barrier()

        def start_read(j, slot):
            base = wid * EPW + j * CH
            pltpu.async_copy(ms.at[pl.ds(base, CH)], bv.at[slot], sr.at[slot])

        def wait_read(slot):
            pltpu.make_async_copy(ms.at[pl.ds(0, CH)], bv.at[slot],
                                  sr.at[slot]).wait()

        def start_add(j, slot):
            pltpu.async_copy(bv.at[slot], acc.at[ia.at[j]], sa.at[slot],
                             add=True)

        def wait_add(slot):
            pltpu.make_async_copy(bv.at[slot], acc.at[ia.at[0]],
                                  sa.at[slot]).wait()

        start_read(0, 0)

        def body(j, carry):
            slot = lax.rem(j, 2)
            prev = 1 - slot

            @pl.when(j >= 2)
            def _():
                wait_add(slot)

            start_read(j, slot)
            wait_read(prev)
            start_add(j - 1, prev)
            return carry

        lax.fori_loop(1, NCHUNK, body, 0)
        last = (NCHUNK - 1) % 2
        wait_read(last)
        start_add(NCHUNK - 1, last)
        wait_add(last)
        wait_add(1 - last)
        plsc.subcore_barrier()
        pltpu.sync_copy(acc.at[pl.ds(sid * TR, TR)],
                        out.at[cid, pl.ds(sid * TR, TR)])

    return sk


def _sc_gather_pair(t1, t2, idx1, idx2):
    return _gather_pair_kernel(t1.shape[1])(t1, t2, idx1, idx2)


def _sc_scatter_add(ms, didx, zb):
    return _scatter_add_kernel()(ms, didx, zb)


def _sigmoid(x):
    return 1.0 / (1.0 + jnp.exp(-x))


def _silu(x):
    return x * _sigmoid(x)


def _pos_stats(pos, batch_rows):
    def body(pos_ref, bat_ref, ps_ref, cnt_ref):
        @pl.when(pl.program_id(0) == 0)
        def _():
            ps_ref[...] = jnp.zeros_like(ps_ref)
            cnt_ref[...] = jnp.zeros_like(cnt_ref)

        bat = bat_ref[...].reshape(1, NB)
        oh = (lax.broadcasted_iota(jnp.int32, (NG, NB), 0) == bat).astype(F32)
        ps_ref[...] += oh @ pos_ref[...]
        cnt_ref[...] += jnp.sum(oh, axis=1, keepdims=True)

    return pl.pallas_call(
        body,
        grid=(NBLK,),
        in_specs=[pl.BlockSpec((NB, 3), lambda i: (i, 0)),
                  pl.BlockSpec((1, 1, NB), lambda i: (i, 0, 0))],
        out_specs=(pl.BlockSpec((NG, 3), lambda i: (0, 0)),
                   pl.BlockSpec((NG, 1), lambda i: (0, 0))),
        out_shape=(jax.ShapeDtypeStruct((NG, 3), F32),
                   jax.ShapeDtypeStruct((NG, 1), F32)),
    )(pos, batch_rows)


def _init_nodes(x, pos, batch_col, ps, cnt, W_atom, b_atom, W1a, W1b):
    def body(x_ref, pos_ref, bat_ref, ps_ref, cnt_ref, wa_ref, ba_ref,
             wp_ref, wq_ref, s_ref, p_ref, q_ref, pp_ref):
        mean = ps_ref[...] / jnp.maximum(cnt_ref[...], 1.0)
        oh = (lax.broadcasted_iota(jnp.int32, (NB, NG), 1)
              == bat_ref[...]).astype(F32)
        cpos = pos_ref[...] - oh @ mean
        pp_ref[...] = jnp.concatenate(
            [cpos, jnp.zeros((NB, SDIM - 3), F32)], axis=1)
        s0 = x_ref[...] @ wa_ref[...] + ba_ref[...]
        s_ref[...] = s0
        p_ref[...] = s0 @ wp_ref[...]
        q_ref[...] = s0 @ wq_ref[...]

    return pl.pallas_call(
        body,
        grid=(NBLK,),
        in_specs=[
            pl.BlockSpec((NB, 16), lambda i: (i, 0)),
            pl.BlockSpec((NB, 3), lambda i: (i, 0)),
            pl.BlockSpec((NB, 1), lambda i: (i, 0)),
            pl.BlockSpec((NG, 3), lambda i: (0, 0)),
            pl.BlockSpec((NG, 1), lambda i: (0, 0)),
            pl.BlockSpec((16, SDIM), lambda i: (0, 0)),
            pl.BlockSpec((1, SDIM), lambda i: (0, 0)),
            pl.BlockSpec((SDIM, SDIM), lambda i: (0, 0)),
            pl.BlockSpec((SDIM, SDIM), lambda i: (0, 0)),
        ],
        out_specs=(pl.BlockSpec((NB, SDIM), lambda i: (i, 0)),
                   pl.BlockSpec((NB, SDIM), lambda i: (i, 0)),
                   pl.BlockSpec((NB, SDIM), lambda i: (i, 0)),
                   pl.BlockSpec((NB, SDIM), lambda i: (i, 0))),
        out_shape=(jax.ShapeDtypeStruct((N, SDIM), F32),
                   jax.ShapeDtypeStruct((N, SDIM), F32),
                   jax.ShapeDtypeStruct((N, SDIM), F32),
                   jax.ShapeDtypeStruct((N, SDIM), F32)),
    )(x, pos, batch_col, ps, cnt, W_atom, b_atom, W1a, W1b)


def _edge_geometry(posS, posD):
    def body(ps_ref, pd_ref, out_ref):
        s3 = ps_ref[...][:, 0:3]
        d3 = pd_ref[...][:, 0:3]
        r = d3 - s3
        d2 = jnp.sum(r * r, axis=1, keepdims=True)
        aa = jnp.sum(d3 * s3, axis=1, keepdims=True)
        dd = jnp.sqrt(jnp.maximum(d2, 1e-6))
        step = CUTOFF / (RBF - 1)
        centers = lax.broadcasted_iota(jnp.int32, (1, RBF), 1).astype(F32) * step
        gamma = (RBF / CUTOFF) ** 2
        rbf = jnp.exp(-gamma * (dd - centers) ** 2)
        out_ref[...] = jnp.concatenate(
            [rbf, aa, jnp.zeros((BE, 7), F32)], axis=1)

    return pl.pallas_call(
        body,
        grid=(NEB,),
        in_specs=[pl.BlockSpec((BE, SDIM), lambda i: (i, 0)),
                  pl.BlockSpec((BE, SDIM), lambda i: (i, 0))],
        out_specs=pl.BlockSpec((BE, 40), lambda i: (i, 0)),
        out_shape=jax.ShapeDtypeStruct((E_PAD, 40), F32),
    )(posS, posD)


def _edge_mlp(gs, gd, rbfa, W1cd, b1, W2s, b2s):
    def body(gs_ref, gd_ref, rb_ref, w1_ref, b1_ref, w2_ref, b2_ref, out_ref):
        pre = gs_ref[...] + gd_ref[...] + rb_ref[...] @ w1_ref[...] + b1_ref[...]
        h = _silu(pre)
        m = h @ w2_ref[...] + b2_ref[...]
        row = (pl.program_id(0) * BE
               + lax.broadcasted_iota(jnp.int32, (BE, 1), 0))
        out_ref[...] = jnp.where(row < E, m, 0.0)

    return pl.pallas_call(
        body,
        grid=(NEB,),
        in_specs=[
            pl.BlockSpec((BE, SDIM), lambda i: (i, 0)),
            pl.BlockSpec((BE, SDIM), lambda i: (i, 0)),
            pl.BlockSpec((BE, 40), lambda i: (i, 0)),
            pl.BlockSpec((40, SDIM), lambda i: (0, 0)),
            pl.BlockSpec((1, SDIM), lambda i: (0, 0)),
            pl.BlockSpec((SDIM, SDIM), lambda i: (0, 0)),
            pl.BlockSpec((1, SDIM), lambda i: (0, 0)),
        ],
        out_specs=pl.BlockSpec((BE, SDIM), lambda i: (i, 0)),
        out_shape=jax.ShapeDtypeStruct((E_PAD, SDIM), F32),
    )(gs, gd, rbfa, W1cd, b1, W2s, b2s)


def _node_update(s, agg0, agg1, Wu, bu, g, bb, Wp, Wq):
    def body(s_ref, a0_ref, a1_ref, wu_ref, bu_ref, g_ref, bb_ref,
             wp_ref, wq_ref, sn_ref, p_ref, q_ref):
        u = (s_ref[...] + (a0_ref[...] + a1_ref[...]) @ wu_ref[...]
             + bu_ref[...])
        mu = jnp.mean(u, axis=1, keepdims=True)
        var = jnp.mean((u - mu) ** 2, axis=1, keepdims=True)
        sn = (u - mu) / jnp.sqrt(var + 1e-5) * g_ref[...] + bb_ref[...]
        sn_ref[...] = sn
        p_ref[...] = sn @ wp_ref[...]
        q_ref[...] = sn @ wq_ref[...]

    return pl.pallas_call(
        body,
        grid=(NBLK,),
        in_specs=[
            pl.BlockSpec((NB, SDIM), lambda i: (i, 0)),
            pl.BlockSpec((NB, SDIM), lambda i: (i, 0)),
            pl.BlockSpec((NB, SDIM), lambda i: (i, 0)),
            pl.BlockSpec((SDIM, SDIM), lambda i: (0, 0)),
            pl.BlockSpec((1, SDIM), lambda i: (0, 0)),
            pl.BlockSpec((1, SDIM), lambda i: (0, 0)),
            pl.BlockSpec((1, SDIM), lambda i: (0, 0)),
            pl.BlockSpec((SDIM, SDIM), lambda i: (0, 0)),
            pl.BlockSpec((SDIM, SDIM), lambda i: (0, 0)),
        ],
        out_specs=(pl.BlockSpec((NB, SDIM), lambda i: (i, 0)),
                   pl.BlockSpec((NB, SDIM), lambda i: (i, 0)),
                   pl.BlockSpec((NB, SDIM), lambda i: (i, 0))),
        out_shape=(jax.ShapeDtypeStruct((N, SDIM), F32),
                   jax.ShapeDtypeStruct((N, SDIM), F32),
                   jax.ShapeDtypeStruct((N, SDIM), F32)),
    )(s, agg0, agg1, Wu, bu, g, bb, Wp, Wq)


def _head_mlps(s, W_lat, b_lat, Wg1, bg1, Wg2, bg2, Wn1, bn1, Wn2, bn2):
    def body(s_ref, wl, bl, wg1, bg1_, wg2, bg2_, wn1, bn1_, wn2, bn2_,
             gl_ref, node_ref):
        out = s_ref[...] @ wl[...] + bl[...]
        hg = _silu(out @ wg1[...] + bg1_[...])
        gl_ref[...] = hg @ wg2[...] + bg2_[...]
        hn = _silu(out @ wn1[...] + bn1_[...])
        node_ref[...] = hn @ wn2[...] + bn2_[...]

    return pl.pallas_call(
        body,
        grid=(NBLK,),
        in_specs=[
            pl.BlockSpec((NB, SDIM), lambda i: (i, 0)),
            pl.BlockSpec((SDIM, LATENT), lambda i: (0, 0)),
            pl.BlockSpec((1, LATENT), lambda i: (0, 0)),
            pl.BlockSpec((LATENT, LATENT), lambda i: (0, 0)),
            pl.BlockSpec((1, LATENT), lambda i: (0, 0)),
            pl.BlockSpec((LATENT, 1), lambda i: (0, 0)),
            pl.BlockSpec((1, 1), lambda i: (0, 0)),
            pl.BlockSpec((LATENT, LATENT), lambda i: (0, 0)),
            pl.BlockSpec((1, LATENT), lambda i: (0, 0)),
            pl.BlockSpec((LATENT, LATENT), lambda i: (0, 0)),
            pl.BlockSpec((1, LATENT), lambda i: (0, 0)),
        ],
        out_specs=(pl.BlockSpec((NB, 1), lambda i: (i, 0)),
                   pl.BlockSpec((NB, LATENT), lambda i: (i, 0))),
        out_shape=(jax.ShapeDtypeStruct((N, 1), F32),
                   jax.ShapeDtypeStruct((N, LATENT), F32)),
    )(s, W_lat, b_lat, Wg1, bg1, Wg2, bg2, Wn1, bn1, Wn2, bn2)


def _gate_max(gl, batch_col):
    def body(gl_ref, bat_ref, gm_ref):
        @pl.when(pl.program_id(0) == 0)
        def _():
            gm_ref[...] = jnp.full_like(gm_ref, -jnp.inf)

        oh = (lax.broadcasted_iota(jnp.int32, (NB, NG), 1) == bat_ref[...])
        masked = jnp.where(oh, gl_ref[...], -jnp.inf)
        gm_ref[...] = jnp.maximum(gm_ref[...],
                                  jnp.max(masked, axis=0, keepdims=True))

    return pl.pallas_call(
        body,
        grid=(NBLK,),
        in_specs=[pl.BlockSpec((NB, 1), lambda i: (i, 0)),
                  pl.BlockSpec((NB, 1), lambda i: (i, 0))],
        out_specs=pl.BlockSpec((1, NG), lambda i: (0, 0)),
        out_shape=jax.ShapeDtypeStruct((1, NG), F32),
    )(gl, batch_col)


def _pool(gl, node, batch_col, batch_rows, gmax):
    def body(gl_ref, node_ref, bat_ref, batr_ref, gm_ref, num_ref, gs_ref):
        @pl.when(pl.program_id(0) == 0)
        def _():
            num_ref[...] = jnp.zeros_like(num_ref)
            gs_ref[...] = jnp.zeros_like(gs_ref)

        oh = (lax.broadcasted_iota(jnp.int32, (NB, NG), 1) == bat_ref[...])
        gmax_g = jnp.sum(jnp.where(oh, gm_ref[...], 0.0), axis=1,
                         keepdims=True)
        eg = jnp.exp(gl_ref[...] - gmax_g)
        batr = batr_ref[...].reshape(1, NB)
        ohT = (lax.broadcasted_iota(jnp.int32, (NG, NB), 0)
               == batr).astype(F32)
        num_ref[...] += ohT @ (eg * node_ref[...])
        gs_ref[...] += ohT @ eg

    return pl.pallas_call(
        body,
        grid=(NBLK,),
        in_specs=[
            pl.BlockSpec((NB, 1), lambda i: (i, 0)),
            pl.BlockSpec((NB, LATENT), lambda i: (i, 0)),
            pl.BlockSpec((NB, 1), lambda i: (i, 0)),
            pl.BlockSpec((1, 1, NB), lambda i: (i, 0, 0)),
            pl.BlockSpec((1, NG), lambda i: (0, 0)),
        ],
        out_specs=(pl.BlockSpec((NG, LATENT), lambda i: (0, 0)),
                   pl.BlockSpec((NG, 1), lambda i: (0, 0))),
        out_shape=(jax.ShapeDtypeStruct((NG, LATENT), F32),
                   jax.ShapeDtypeStruct((NG, 1), F32)),
    )(gl, node, batch_col, batch_rows, gmax)


def _finalize(num, gs):
    def body(num_ref, gs_ref, out_ref):
        out_ref[...] = num_ref[...] / (gs_ref[...] + 1e-16)

    return pl.pallas_call(
        body,
        in_specs=[pl.BlockSpec((NG, LATENT), lambda: (0, 0)),
                  pl.BlockSpec((NG, 1), lambda: (0, 0))],
        out_specs=pl.BlockSpec((NG, LATENT), lambda: (0, 0)),
        out_shape=jax.ShapeDtypeStruct((NG, LATENT), F32),
    )(num, gs)


def _prep_idx(idx):
    p = jnp.zeros((E_PAD,), jnp.int32).at[:E].set(idx)
    return p.reshape(NW, NCHUNK, CH)


def kernel(x, pos, edge_index_local, edge_index_global, batch, W_atom,
           b_atom, W1, b1, W2, b2, W_upd, b_upd, ln_g, ln_b, W_lat, b_lat,
           Wn1, bn1, Wn2, bn2, Wg1, bg1, Wg2, bg2):
    batch_col = batch.reshape(N, 1)
    batch_rows = batch.reshape(NBLK, 1, NB)
    row = lambda v: v.reshape(1, -1)

    srcR = [_prep_idx(edge_index_local[0]), _prep_idx(edge_index_global[0])]
    dstR = [_prep_idx(edge_index_local[1]), _prep_idx(edge_index_global[1])]

    ps, cnt = _pos_stats(pos, batch_rows)
    s, P, Q, pos128 = _init_nodes(
        x, pos, batch_col, ps, cnt, W_atom, row(b_atom),
        W1[0, 0, :SDIM, :], W1[0, 0, SDIM:2 * SDIM, :])

    rbfa = []
    for j in range(2):
        posS, posD = _sc_gather_pair(pos128, pos128, srcR[j], dstR[j])
        rbfa.append(_edge_geometry(posS, posD))

    zb = jnp.zeros((TR, SDIM), F32)

    for p in range(2 * LAYERS):
        l, j = p // 2, p % 2
        W1cd = jnp.zeros((40, SDIM), F32).at[:33].set(W1[l, j, 2 * SDIM:, :])
        gs, gd = _sc_gather_pair(P, Q, srcR[j], dstR[j])
        ms = _edge_mlp(gs, gd, rbfa[j], W1cd, row(b1[l, j]),
                       W2[l, j][:, :SDIM], row(b2[l, j][:SDIM]))
        agg = _sc_scatter_add(ms, dstR[j], zb)
        a0, a1 = agg[0, :N], agg[1, :N]
        ln_, jn_ = (p + 1) // 2, (p + 1) % 2
        if p == 2 * LAYERS - 1:
            ln_, jn_ = 0, 0
        s, P, Q = _node_update(
            s, a0, a1, W_upd[l, j], row(b_upd[l, j]),
            row(ln_g[l, j]), row(ln_b[l, j]),
            W1[ln_, jn_, :SDIM, :], W1[ln_, jn_, SDIM:2 * SDIM, :])

    gl, node = _head_mlps(s, W_lat, row(b_lat), Wg1, row(bg1),
                          Wg2, bg2.reshape(1, 1), Wn1, row(bn1),
                          Wn2, row(bn2))
    gmax = _gate_max(gl, batch_col)
    num, gs_ = _pool(gl, node, batch_col, batch_rows, gmax)
    return _finalize(num, gs_)


# speedup vs baseline: 2.2795x; 1.1631x over previous
import functools

import jax
import jax.numpy as jnp
from jax import lax
from jax.experimental import pallas as pl
from jax.experimental.pallas import tpu as pltpu
from jax.experimental.pallas import tpu_sc as plsc

F32 = jnp.float32
N = 10000
NG = 256
SDIM = 128
RBF = 32
LATENT = 128
CUTOFF = 7.5
LAYERS = 5
E = 320000

NCORE = 2
NSUB = 16
NW = NCORE * NSUB
CH = 128
EPW = 10240
NCHUNK = EPW // CH
E_PAD = NW * EPW
N_PAD = 10240
TR = N_PAD // NSUB

NB = 1000
NBLK = N // NB
BE = 2048
NEB = E_PAD // BE

@functools.lru_cache(maxsize=None)
def _mesh():
    return plsc.VectorSubcoreMesh(core_axis_name="c", subcore_axis_name="s",
                                  num_cores=NCORE, num_subcores=NSUB)


@functools.lru_cache(maxsize=None)
def _gather_add_kernel():
    D = SDIM

    @functools.partial(
        pl.kernel,
        mesh=_mesh(),
        out_type=jax.ShapeDtypeStruct((E_PAD, D), F32),
        scratch_types=[
            pltpu.VMEM((NCHUNK, CH), jnp.int32),
            pltpu.VMEM((NCHUNK, CH), jnp.int32),
            pltpu.VMEM((2, CH, D), F32),
            pltpu.VMEM((2, CH, D), F32),
            pltpu.SemaphoreType.DMA((2,)),
            pltpu.SemaphoreType.DMA((2,)),
            pltpu.SemaphoreType.DMA((2,)),
        ],
        name="sc_gather_add",
    )
    def gk(t1, t2, idx1, idx2, o1, i1a, i2a, b1, b2, sg1, sg2, sw1):
        wid = lax.axis_index("s") * NCORE + lax.axis_index("c")
        pltpu.sync_copy(idx1.at[wid], i1a)
        pltpu.sync_copy(idx2.at[wid], i2a)

        def start_gather(j, slot):
            pltpu.async_copy(t1.at[i1a.at[j]], b1.at[slot], sg1.at[slot])
            pltpu.async_copy(t2.at[i2a.at[j]], b2.at[slot], sg2.at[slot])

        def wait_gather(slot):
            pltpu.make_async_copy(t1.at[pl.ds(0, CH)], b1.at[slot],
                                  sg1.at[slot]).wait()
            pltpu.make_async_copy(t2.at[pl.ds(0, CH)], b2.at[slot],
                                  sg2.at[slot]).wait()

        def add_rows(slot):
            def rowbody(r, carry):
                for k in range(D // 16):
                    sl = pl.ds(k * 16, 16)
                    b1[slot, r, sl] = b1[slot, r, sl] + b2[slot, r, sl]
                return carry

            lax.fori_loop(0, CH, rowbody, 0)

        def start_write(j, slot):
            base = wid * EPW + j * CH
            pltpu.async_copy(b1.at[slot], o1.at[pl.ds(base, CH)], sw1.at[slot])

        def wait_write(slot):
            pltpu.make_async_copy(b1.at[slot], o1.at[pl.ds(0, CH)],
                                  sw1.at[slot]).wait()

        start_gather(0, 0)

        def body(j, carry):
            slot = lax.rem(j, 2)
            prev = 1 - slot

            @pl.when(j >= 2)
            def _():
                wait_write(slot)

            start_gather(j, slot)
            wait_gather(prev)
            add_rows(prev)
            start_write(j - 1, prev)
            return carry

        lax.fori_loop(1, NCHUNK, body, 0)
        last = (NCHUNK - 1) % 2
        wait_gather(last)
        add_rows(last)
        start_write(NCHUNK - 1, last)
        wait_write(last)
        wait_write(1 - last)

    return gk


@functools.lru_cache(maxsize=None)
def _scatter_add_kernel():
    @functools.partial(
        pl.kernel,
        mesh=_mesh(),
        out_type=jax.ShapeDtypeStruct((NCORE, N_PAD, SDIM), F32),
        scratch_types=[
            pltpu.VMEM((NCHUNK, CH), jnp.int32),
            pltpu.VMEM((2, CH, SDIM), F32),
            pltpu.VMEM_SHARED((N_PAD, SDIM), F32),
            pltpu.SemaphoreType.DMA((2,)),
            pltpu.SemaphoreType.DMA((2,)),
        ],
        name="sc_scatter_add",
    )
    def sk(ms, didx, zb, out, ia, bv, acc, sr, sa):
        cid = lax.axis_index("c")
        sid = lax.axis_index("s")
        wid = sid * NCORE + cid
        pltpu.sync_copy(zb, acc.at[pl.ds(sid * TR, TR)])
        pltpu.sync_copy(didx.at[wid], ia)
        plsc.subcore_barrier()

        def start_read(j, slot):
            base = wid * EPW + j * CH
            pltpu.async_copy(ms.at[pl.ds(base, CH)], bv.at[slot], sr.at[slot])

        def wait_read(slot):
            pltpu.make_async_copy(ms.at[pl.ds(0, CH)], bv.at[slot],
                                  sr.at[slot]).wait()

        def start_add(j, slot):
            pltpu.async_copy(bv.at[slot], acc.at[ia.at[j]], sa.at[slot],
                             add=True)

        def wait_add(slot):
            pltpu.make_async_copy(bv.at[slot], acc.at[ia.at[0]],
                                  sa.at[slot]).wait()

        start_read(0, 0)

        def body(j, carry):
            slot = lax.rem(j, 2)
            prev = 1 - slot

            @pl.when(j >= 2)
            def _():
                wait_add(slot)

            start_read(j, slot)
            wait_read(prev)
            start_add(j - 1, prev)
            return carry

        lax.fori_loop(1, NCHUNK, body, 0)
        last = (NCHUNK - 1) % 2
        wait_read(last)
        start_add(NCHUNK - 1, last)
        wait_add(last)
        wait_add(1 - last)
        plsc.subcore_barrier()
        pltpu.sync_copy(acc.at[pl.ds(sid * TR, TR)],
                        out.at[cid, pl.ds(sid * TR, TR)])

    return sk


def _sc_gather_add(t1, t2, idx1, idx2):
    return _gather_add_kernel()(t1, t2, idx1, idx2)


def _sc_scatter_add(ms, didx, zb):
    return _scatter_add_kernel()(ms, didx, zb)


def _sigmoid(x):
    return 1.0 / (1.0 + jnp.exp(-x))


def _silu(x):
    return x * _sigmoid(x)


def _pos_stats(pos, batch_rows):
    def body(pos_ref, bat_ref, ps_ref, cnt_ref):
        @pl.when(pl.program_id(0) == 0)
        def _():
            ps_ref[...] = jnp.zeros_like(ps_ref)
            cnt_ref[...] = jnp.zeros_like(cnt_ref)

        bat = bat_ref[...].reshape(1, NB)
        oh = (lax.broadcasted_iota(jnp.int32, (NG, NB), 0) == bat).astype(F32)
        ps_ref[...] += oh @ pos_ref[...]
        cnt_ref[...] += jnp.sum(oh, axis=1, keepdims=True)

    return pl.pallas_call(
        body,
        grid=(NBLK,),
        in_specs=[pl.BlockSpec((NB, 3), lambda i: (i, 0)),
                  pl.BlockSpec((1, 1, NB), lambda i: (i, 0, 0))],
        out_specs=(pl.BlockSpec((NG, 3), lambda i: (0, 0)),
                   pl.BlockSpec((NG, 1), lambda i: (0, 0))),
        out_shape=(jax.ShapeDtypeStruct((NG, 3), F32),
                   jax.ShapeDtypeStruct((NG, 1), F32)),
    )(pos, batch_rows)


def _init_nodes(x, pos, batch_col, ps, cnt, W_atom, b_atom, W1a, W1b):
    def body(x_ref, pos_ref, bat_ref, ps_ref, cnt_ref, wa_ref, ba_ref,
             wp_ref, wq_ref, s_ref, p_ref, q_ref, pa_ref, pb_ref):
        mean = ps_ref[...] / jnp.maximum(cnt_ref[...], 1.0)
        oh = (lax.broadcasted_iota(jnp.int32, (NB, NG), 1)
              == bat_ref[...]).astype(F32)
        cpos = pos_ref[...] - oh @ mean
        sq = jnp.sum(cpos * cpos, axis=1, keepdims=True)
        pad = jnp.zeros((NB, SDIM - 4), F32)
        pa_ref[...] = jnp.concatenate([-cpos, sq, pad], axis=1)
        pb_ref[...] = jnp.concatenate([cpos, sq, pad], axis=1)
        s0 = x_ref[...] @ wa_ref[...] + ba_ref[...]
        s_ref[...] = s0
        p_ref[...] = s0 @ wp_ref[...]
        q_ref[...] = s0 @ wq_ref[...]

    return pl.pallas_call(
        body,
        grid=(NBLK,),
        in_specs=[
            pl.BlockSpec((NB, 16), lambda i: (i, 0)),
            pl.BlockSpec((NB, 3), lambda i: (i, 0)),
            pl.BlockSpec((NB, 1), lambda i: (i, 0)),
            pl.BlockSpec((NG, 3), lambda i: (0, 0)),
            pl.BlockSpec((NG, 1), lambda i: (0, 0)),
            pl.BlockSpec((16, SDIM), lambda i: (0, 0)),
            pl.BlockSpec((1, SDIM), lambda i: (0, 0)),
            pl.BlockSpec((SDIM, SDIM), lambda i: (0, 0)),
            pl.BlockSpec((SDIM, SDIM), lambda i: (0, 0)),
        ],
        out_specs=(pl.BlockSpec((NB, SDIM), lambda i: (i, 0)),
                   pl.BlockSpec((NB, SDIM), lambda i: (i, 0)),
                   pl.BlockSpec((NB, SDIM), lambda i: (i, 0)),
                   pl.BlockSpec((NB, SDIM), lambda i: (i, 0)),
                   pl.BlockSpec((NB, SDIM), lambda i: (i, 0))),
        out_shape=(jax.ShapeDtypeStruct((N, SDIM), F32),
                   jax.ShapeDtypeStruct((N, SDIM), F32),
                   jax.ShapeDtypeStruct((N, SDIM), F32),
                   jax.ShapeDtypeStruct((N, SDIM), F32),
                   jax.ShapeDtypeStruct((N, SDIM), F32)),
    )(x, pos, batch_col, ps, cnt, W_atom, b_atom, W1a, W1b)


def _edge_geometry(gpos):
    def body(g_ref, out_ref):
        g = g_ref[...]
        r = g[:, 0:3]
        ssum = g[:, 3:4]
        d2 = jnp.sum(r * r, axis=1, keepdims=True)
        aa = (ssum - d2) * 0.5
        dd = jnp.sqrt(jnp.maximum(d2, 1e-6))
        step = CUTOFF / (RBF - 1)
        centers = lax.broadcasted_iota(jnp.int32, (1, RBF), 1).astype(F32) * step
        gamma = (RBF / CUTOFF) ** 2
        rbf = jnp.exp(-gamma * (dd - centers) ** 2)
        out_ref[...] = jnp.concatenate(
            [rbf, aa, jnp.zeros((BE, 7), F32)], axis=1)

    return pl.pallas_call(
        body,
        grid=(NEB,),
        in_specs=[pl.BlockSpec((BE, SDIM), lambda i: (i, 0))],
        out_specs=pl.BlockSpec((BE, 40), lambda i: (i, 0)),
        out_shape=jax.ShapeDtypeStruct((E_PAD, 40), F32),
    )(gpos)


def _edge_mlp(g, rbfa, W1cd, b1, W2s, b2s):
    def body(g_ref, rb_ref, w1_ref, b1_ref, w2_ref, b2_ref, out_ref):
        pre = g_ref[...] + rb_ref[...] @ w1_ref[...] + b1_ref[...]
        h = _silu(pre)
        m = h @ w2_ref[...] + b2_ref[...]
        row = (pl.program_id(0) * BE
               + lax.broadcasted_iota(jnp.int32, (BE, 1), 0))
        out_ref[...] = jnp.where(row < E, m, 0.0)

    return pl.pallas_call(
        body,
        grid=(NEB,),
        in_specs=[
            pl.BlockSpec((BE, SDIM), lambda i: (i, 0)),
            pl.BlockSpec((BE, 40), lambda i: (i, 0)),
            pl.BlockSpec((40, SDIM), lambda i: (0, 0)),
            pl.BlockSpec((1, SDIM), lambda i: (0, 0)),
            pl.BlockSpec((SDIM, SDIM), lambda i: (0, 0)),
            pl.BlockSpec((1, SDIM), lambda i: (0, 0)),
        ],
        out_specs=pl.BlockSpec((BE, SDIM), lambda i: (i, 0)),
        out_shape=jax.ShapeDtypeStruct((E_PAD, SDIM), F32),
    )(g, rbfa, W1cd, b1, W2s, b2s)


def _node_update(s, agg0, agg1, Wu, bu, g, bb, Wp, Wq):
    def body(s_ref, a0_ref, a1_ref, wu_ref, bu_ref, g_ref, bb_ref,
             wp_ref, wq_ref, sn_ref, p_ref, q_ref):
        u = (s_ref[...] + (a0_ref[...] + a1_ref[...]) @ wu_ref[...]
             + bu_ref[...])
        mu = jnp.mean(u, axis=1, keepdims=True)
        var = jnp.mean((u - mu) ** 2, axis=1, keepdims=True)
        sn = (u - mu) / jnp.sqrt(var + 1e-5) * g_ref[...] + bb_ref[...]
        sn_ref[...] = sn
        p_ref[...] = sn @ wp_ref[...]
        q_ref[...] = sn @ wq_ref[...]

    return pl.pallas_call(
        body,
        grid=(NBLK,),
        in_specs=[
            pl.BlockSpec((NB, SDIM), lambda i: (i, 0)),
            pl.BlockSpec((NB, SDIM), lambda i: (i, 0)),
            pl.BlockSpec((NB, SDIM), lambda i: (i, 0)),
            pl.BlockSpec((SDIM, SDIM), lambda i: (0, 0)),
            pl.BlockSpec((1, SDIM), lambda i: (0, 0)),
            pl.BlockSpec((1, SDIM), lambda i: (0, 0)),
            pl.BlockSpec((1, SDIM), lambda i: (0, 0)),
            pl.BlockSpec((SDIM, SDIM), lambda i: (0, 0)),
            pl.BlockSpec((SDIM, SDIM), lambda i: (0, 0)),
        ],
        out_specs=(pl.BlockSpec((NB, SDIM), lambda i: (i, 0)),
                   pl.BlockSpec((NB, SDIM), lambda i: (i, 0)),
                   pl.BlockSpec((NB, SDIM), lambda i: (i, 0))),
        out_shape=(jax.ShapeDtypeStruct((N, SDIM), F32),
                   jax.ShapeDtypeStruct((N, SDIM), F32),
                   jax.ShapeDtypeStruct((N, SDIM), F32)),
    )(s, agg0, agg1, Wu, bu, g, bb, Wp, Wq)


def _head_mlps(s, W_lat, b_lat, Wg1, bg1, Wg2, bg2, Wn1, bn1, Wn2, bn2):
    def body(s_ref, wl, bl, wg1, bg1_, wg2, bg2_, wn1, bn1_, wn2, bn2_,
             gl_ref, node_ref):
        out = s_ref[...] @ wl[...] + bl[...]
        hg = _silu(out @ wg1[...] + bg1_[...])
        gl_ref[...] = hg @ wg2[...] + bg2_[...]
        hn = _silu(out @ wn1[...] + bn1_[...])
        node_ref[...] = hn @ wn2[...] + bn2_[...]

    return pl.pallas_call(
        body,
        grid=(NBLK,),
        in_specs=[
            pl.BlockSpec((NB, SDIM), lambda i: (i, 0)),
            pl.BlockSpec((SDIM, LATENT), lambda i: (0, 0)),
            pl.BlockSpec((1, LATENT), lambda i: (0, 0)),
            pl.BlockSpec((LATENT, LATENT), lambda i: (0, 0)),
            pl.BlockSpec((1, LATENT), lambda i: (0, 0)),
            pl.BlockSpec((LATENT, 1), lambda i: (0, 0)),
            pl.BlockSpec((1, 1), lambda i: (0, 0)),
            pl.BlockSpec((LATENT, LATENT), lambda i: (0, 0)),
            pl.BlockSpec((1, LATENT), lambda i: (0, 0)),
            pl.BlockSpec((LATENT, LATENT), lambda i: (0, 0)),
            pl.BlockSpec((1, LATENT), lambda i: (0, 0)),
        ],
        out_specs=(pl.BlockSpec((NB, 1), lambda i: (i, 0)),
                   pl.BlockSpec((NB, LATENT), lambda i: (i, 0))),
        out_shape=(jax.ShapeDtypeStruct((N, 1), F32),
                   jax.ShapeDtypeStruct((N, LATENT), F32)),
    )(s, W_lat, b_lat, Wg1, bg1, Wg2, bg2, Wn1, bn1, Wn2, bn2)


def _gate_max(gl, batch_col):
    def body(gl_ref, bat_ref, gm_ref):
        @pl.when(pl.program_id(0) == 0)
        def _():
            gm_ref[...] = jnp.full_like(gm_ref, -jnp.inf)

        oh = (lax.broadcasted_iota(jnp.int32, (NB, NG), 1) == bat_ref[...])
        masked = jnp.where(oh, gl_ref[...], -jnp.inf)
        gm_ref[...] = jnp.maximum(gm_ref[...],
                                  jnp.max(masked, axis=0, keepdims=True))

    return pl.pallas_call(
        body,
        grid=(NBLK,),
        in_specs=[pl.BlockSpec((NB, 1), lambda i: (i, 0)),
                  pl.BlockSpec((NB, 1), lambda i: (i, 0))],
        out_specs=pl.BlockSpec((1, NG), lambda i: (0, 0)),
        out_shape=jax.ShapeDtypeStruct((1, NG), F32),
    )(gl, batch_col)


def _pool(gl, node, batch_col, batch_rows, gmax):
    def body(gl_ref, node_ref, bat_ref, batr_ref, gm_ref, num_ref, gs_ref):
        @pl.when(pl.program_id(0) == 0)
        def _():
            num_ref[...] = jnp.zeros_like(num_ref)
            gs_ref[...] = jnp.zeros_like(gs_ref)

        oh = (lax.broadcasted_iota(jnp.int32, (NB, NG), 1) == bat_ref[...])
        gmax_g = jnp.sum(jnp.where(oh, gm_ref[...], 0.0), axis=1,
                         keepdims=True)
        eg = jnp.exp(gl_ref[...] - gmax_g)
        batr = batr_ref[...].reshape(1, NB)
        ohT = (lax.broadcasted_iota(jnp.int32, (NG, NB), 0)
               == batr).astype(F32)
        num_ref[...] += ohT @ (eg * node_ref[...])
        gs_ref[...] += ohT @ eg

    return pl.pallas_call(
        body,
        grid=(NBLK,),
        in_specs=[
            pl.BlockSpec((NB, 1), lambda i: (i, 0)),
            pl.BlockSpec((NB, LATENT), lambda i: (i, 0)),
            pl.BlockSpec((NB, 1), lambda i: (i, 0)),
            pl.BlockSpec((1, 1, NB), lambda i: (i, 0, 0)),
            pl.BlockSpec((1, NG), lambda i: (0, 0)),
        ],
        out_specs=(pl.BlockSpec((NG, LATENT), lambda i: (0, 0)),
                   pl.BlockSpec((NG, 1), lambda i: (0, 0))),
        out_shape=(jax.ShapeDtypeStruct((NG, LATENT), F32),
                   jax.ShapeDtypeStruct((NG, 1), F32)),
    )(gl, node, batch_col, batch_rows, gmax)


def _finalize(num, gs):
    def body(num_ref, gs_ref, out_ref):
        out_ref[...] = num_ref[...] / (gs_ref[...] + 1e-16)

    return pl.pallas_call(
        body,
        in_specs=[pl.BlockSpec((NG, LATENT), lambda: (0, 0)),
                  pl.BlockSpec((NG, 1), lambda: (0, 0))],
        out_specs=pl.BlockSpec((NG, LATENT), lambda: (0, 0)),
        out_shape=jax.ShapeDtypeStruct((NG, LATENT), F32),
    )(num, gs)


def _prep_idx(idx):
    p = jnp.zeros((E_PAD,), jnp.int32).at[:E].set(idx)
    return p.reshape(NW, NCHUNK, CH)


def kernel(x, pos, edge_index_local, edge_index_global, batch, W_atom,
           b_atom, W1, b1, W2, b2, W_upd, b_upd, ln_g, ln_b, W_lat, b_lat,
           Wn1, bn1, Wn2, bn2, Wg1, bg1, Wg2, bg2):
    batch_col = batch.reshape(N, 1)
    batch_rows = batch.reshape(NBLK, 1, NB)
    row = lambda v: v.reshape(1, -1)

    srcR = [_prep_idx(edge_index_local[0]), _prep_idx(edge_index_global[0])]
    dstR = [_prep_idx(edge_index_local[1]), _prep_idx(edge_index_global[1])]

    ps, cnt = _pos_stats(pos, batch_rows)
    s, P, Q, posA, posB = _init_nodes(
        x, pos, batch_col, ps, cnt, W_atom, row(b_atom),
        W1[0, 0, :SDIM, :], W1[0, 0, SDIM:2 * SDIM, :])

    rbfa = []
    for j in range(2):
        gpos = _sc_gather_add(posA, posB, srcR[j], dstR[j])
        rbfa.append(_edge_geometry(gpos))

    zb = jnp.zeros((TR, SDIM), F32)

    for p in range(2 * LAYERS):
        l, j = p // 2, p % 2
        W1cd = jnp.zeros((40, SDIM), F32).at[:33].set(W1[l, j, 2 * SDIM:, :])
        g = _sc_gather_add(P, Q, srcR[j], dstR[j])
        ms = _edge_mlp(g, rbfa[j], W1cd, row(b1[l, j]),
                       W2[l, j][:, :SDIM], row(b2[l, j][:SDIM]))
        agg = _sc_scatter_add(ms, dstR[j], zb)
        a0, a1 = agg[0, :N], agg[1, :N]
        ln_, jn_ = (p + 1) // 2, (p + 1) % 2
        if p == 2 * LAYERS - 1:
            ln_, jn_ = 0, 0
        s, P, Q = _node_update(
            s, a0, a1, W_upd[l, j], row(b_upd[l, j]),
            row(ln_g[l, j]), row(ln_b[l, j]),
            W1[ln_, jn_, :SDIM, :], W1[ln_, jn_, SDIM:2 * SDIM, :])

    gl, node = _head_mlps(s, W_lat, row(b_lat), Wg1, row(bg1),
                          Wg2, bg2.reshape(1, 1), Wn1, row(bn1),
                          Wn2, row(bn2))
    gmax = _gate_max(gl, batch_col)
    num, gs_ = _pool(gl, node, batch_col, batch_rows, gmax)
    return _finalize(num, gs_)
